# baseline (speedup 1.0000x reference)
import functools

import numpy as np

import jax
import jax.numpy as jnp
from jax.experimental import pallas as pl
from jax.experimental.pallas import tpu as pltpu

_F = 12
_M = 512


def _round_up(a, m):
    return (a + m - 1) // m * m


def _fused_kernel(x_ref, w1_ref, b1_ref, w2_ref, b2_ref,
                  f1_ref, f1b_ref, f2_ref, f2b_ref, o_ref, *, n_ch):
    bf = jnp.bfloat16
    x2 = x_ref[...]
    m = x2.shape[0]
    zf = jnp.zeros((m, _F), jnp.float32)

    xm = jnp.concatenate([zf, x2[:, :-_F]], axis=1)
    xp = jnp.concatenate([x2[:, _F:], zf], axis=1)

    a0 = a1 = a2 = None
    for c in range(n_ch):
        h1 = jnp.maximum(
            w1_ref[3 * c] * xm + w1_ref[3 * c + 1] * x2
            + w1_ref[3 * c + 2] * xp + b1_ref[c], 0.0)
        if c == 0:
            a0 = w2_ref[0] * h1
            a1 = w2_ref[1] * h1
            a2 = w2_ref[2] * h1
        else:
            a0 = a0 + w2_ref[3 * c] * h1
            a1 = a1 + w2_ref[3 * c + 1] * h1
            a2 = a2 + w2_ref[3 * c + 2] * h1

    y2 = jnp.maximum(
        jnp.concatenate([zf, a0[:, :-_F]], axis=1) + a1
        + jnp.concatenate([a2[:, _F:], zf], axis=1) + b2_ref[0],
        0.0).astype(bf)

    h = jnp.maximum(
        jnp.dot(y2, f1_ref[...], preferred_element_type=jnp.float32)
        + f1b_ref[...], 0.0).astype(bf)
    o_ref[...] = (jnp.dot(h, f2_ref[...], preferred_element_type=jnp.float32)
                  + f2b_ref[...])


def kernel(x, conv1_w, conv1_b, conv2_w, conv2_b, fc1_w, fc1_b, fc2_w, fc2_b):
    N, C, S, F = x.shape
    assert C == 1 and F == _F
    n_ch = conv1_w.shape[0]
    hid = fc1_w.shape[0]
    bf = jnp.bfloat16

    npad = _round_up(max(N, 1), _M)
    nblocks = npad // _M

    xs = x.reshape(N, S * F).astype(jnp.float32)
    if npad != N:
        xs = jnp.pad(xs, ((0, npad - N), (0, 0)))

    w1_k = conv1_w.reshape(-1).astype(jnp.float32)
    b1_k = conv1_b.reshape(-1).astype(jnp.float32)
    w2_k = conv2_w.reshape(-1).astype(jnp.float32)
    b2_k = conv2_b.reshape(-1).astype(jnp.float32)

    eye = jnp.eye(S, dtype=jnp.float32)
    f1 = jnp.kron(eye, fc1_w.T.astype(jnp.float32))
    f1b = jnp.tile(fc1_b.astype(jnp.float32), S).reshape(1, S * hid)

    blend = np.zeros((2 * S, S + 1), np.float32)
    blend[0, 0] = 1.0
    for s in range(1, S):
        blend[2 * s - 1, s] = 0.5
        blend[2 * s, s] = 0.5
    blend[2 * S - 1, S] = 1.0
    blend = jnp.asarray(blend)
    f2 = jnp.kron(eye, fc2_w.T.astype(jnp.float32)) @ blend
    f2b = (jnp.tile(fc2_b.astype(jnp.float32), S) @ blend).reshape(1, S + 1)

    smem = pl.BlockSpec(memory_space=pltpu.MemorySpace.SMEM)
    full = lambda r, c: pl.BlockSpec((r, c), lambda b: (0, 0))

    out = pl.pallas_call(
        functools.partial(_fused_kernel, n_ch=n_ch),
        out_shape=jax.ShapeDtypeStruct((npad, S + 1), jnp.float32),
        grid=(nblocks,),
        in_specs=[
            pl.BlockSpec((_M, S * F), lambda b: (b, 0)),
            smem, smem, smem, smem,
            full(S * F, S * hid),
            full(1, S * hid),
            full(S * hid, S + 1),
            full(1, S + 1),
        ],
        out_specs=pl.BlockSpec((_M, S + 1), lambda b: (b, 0)),
        compiler_params=pltpu.CompilerParams(
            dimension_semantics=("parallel",),
            vmem_limit_bytes=64 * 1024 * 1024),
    )(xs, w1_k, b1_k, w2_k, b2_k, f1.astype(bf), f1b, f2.astype(bf), f2b)

    return out[:N]


# speedup vs baseline: 1.0057x; 1.0057x over previous
import functools

import numpy as np

import jax
import jax.numpy as jnp
from jax.experimental import pallas as pl
from jax.experimental.pallas import tpu as pltpu

_F = 12
_M = 512


def _round_up(a, m):
    return (a + m - 1) // m * m


def _fused_kernel(x_ref, w1_ref, b1_ref, w2_ref, b2_ref,
                  f1_ref, f1b_ref, f2_ref, f2b_ref, o_ref, *, n_ch):
    bf = jnp.bfloat16
    x2 = x_ref[...].astype(jnp.float32)
    m = x2.shape[0]
    zf = jnp.zeros((m, _F), jnp.float32)

    xm = jnp.concatenate([zf, x2[:, :-_F]], axis=1)
    xp = jnp.concatenate([x2[:, _F:], zf], axis=1)

    a0 = a1 = a2 = None
    for c in range(n_ch):
        h1 = jnp.maximum(
            w1_ref[3 * c] * xm + w1_ref[3 * c + 1] * x2
            + w1_ref[3 * c + 2] * xp + b1_ref[c], 0.0)
        if c == 0:
            a0 = w2_ref[0] * h1
            a1 = w2_ref[1] * h1
            a2 = w2_ref[2] * h1
        else:
            a0 = a0 + w2_ref[3 * c] * h1
            a1 = a1 + w2_ref[3 * c + 1] * h1
            a2 = a2 + w2_ref[3 * c + 2] * h1

    y2 = jnp.maximum(
        jnp.concatenate([zf, a0[:, :-_F]], axis=1) + a1
        + jnp.concatenate([a2[:, _F:], zf], axis=1) + b2_ref[0],
        0.0).astype(bf)

    h = jnp.maximum(
        jnp.dot(y2, f1_ref[...], preferred_element_type=jnp.float32)
        + f1b_ref[...], 0.0).astype(bf)
    o_ref[...] = (jnp.dot(h, f2_ref[...], preferred_element_type=jnp.float32)
                  + f2b_ref[...])


def kernel(x, conv1_w, conv1_b, conv2_w, conv2_b, fc1_w, fc1_b, fc2_w, fc2_b):
    N, C, S, F = x.shape
    assert C == 1 and F == _F
    n_ch = conv1_w.shape[0]
    hid = fc1_w.shape[0]
    bf = jnp.bfloat16

    npad = _round_up(max(N, 1), _M)
    nblocks = npad // _M

    xs = x.reshape(N, S * F).astype(bf)
    if npad != N:
        xs = jnp.pad(xs, ((0, npad - N), (0, 0)))

    w1_k = conv1_w.reshape(-1).astype(jnp.float32)
    b1_k = conv1_b.reshape(-1).astype(jnp.float32)
    w2_k = conv2_w.reshape(-1).astype(jnp.float32)
    b2_k = conv2_b.reshape(-1).astype(jnp.float32)

    eye = jnp.eye(S, dtype=jnp.float32)
    f1 = jnp.kron(eye, fc1_w.T.astype(jnp.float32))
    f1b = jnp.tile(fc1_b.astype(jnp.float32), S).reshape(1, S * hid)

    blend = np.zeros((2 * S, S + 1), np.float32)
    blend[0, 0] = 1.0
    for s in range(1, S):
        blend[2 * s - 1, s] = 0.5
        blend[2 * s, s] = 0.5
    blend[2 * S - 1, S] = 1.0
    blend = jnp.asarray(blend)
    f2 = jnp.kron(eye, fc2_w.T.astype(jnp.float32)) @ blend
    f2b = (jnp.tile(fc2_b.astype(jnp.float32), S) @ blend).reshape(1, S + 1)

    smem = pl.BlockSpec(memory_space=pltpu.MemorySpace.SMEM)
    full = lambda r, c: pl.BlockSpec((r, c), lambda b: (0, 0))

    out = pl.pallas_call(
        functools.partial(_fused_kernel, n_ch=n_ch),
        out_shape=jax.ShapeDtypeStruct((npad, S + 1), jnp.float32),
        grid=(nblocks,),
        in_specs=[
            pl.BlockSpec((_M, S * F), lambda b: (b, 0)),
            smem, smem, smem, smem,
            full(S * F, S * hid),
            full(1, S * hid),
            full(S * hid, S + 1),
            full(1, S + 1),
        ],
        out_specs=pl.BlockSpec((_M, S + 1), lambda b: (b, 0)),
        compiler_params=pltpu.CompilerParams(
            dimension_semantics=("parallel",),
            vmem_limit_bytes=64 * 1024 * 1024),
    )(xs, w1_k, b1_k, w2_k, b2_k, f1.astype(bf), f1b, f2.astype(bf), f2b)

    return out[:N]


# speedup vs baseline: 1.0646x; 1.0585x over previous
import functools

import numpy as np

import jax
import jax.numpy as jnp
from jax.experimental import pallas as pl
from jax.experimental.pallas import tpu as pltpu

_F = 12
_M = 512


def _round_up(a, m):
    return (a + m - 1) // m * m


def _fused_kernel(x_ref, w1_ref, b1_ref, t2_ref, b2_ref,
                  f1_ref, f1b_ref, f2_ref, f2b_ref, o_ref, *, n_ch, grp):
    bf = jnp.bfloat16
    x2 = x_ref[...].astype(jnp.float32)
    m = x2.shape[0]
    sf = x2.shape[1]
    zf = jnp.zeros((m, _F), jnp.float32)

    xm = jnp.concatenate([zf, x2[:, :-_F]], axis=1)
    xp = jnp.concatenate([x2[:, _F:], zf], axis=1)

    acc = None
    for g in range(0, n_ch, grp):
        hs = []
        for c in range(g, g + grp):
            hs.append(jnp.maximum(
                w1_ref[3 * c] * xm + w1_ref[3 * c + 1] * x2
                + w1_ref[3 * c + 2] * xp + b1_ref[c], 0.0).astype(bf))
        hg = jnp.concatenate(hs, axis=1)
        p = jnp.dot(hg, t2_ref[g * sf:(g + grp) * sf, :],
                    preferred_element_type=jnp.float32)
        acc = p if g == 0 else acc + p

    y2 = jnp.maximum(acc + b2_ref[0], 0.0).astype(bf)

    h = jnp.maximum(
        jnp.dot(y2, f1_ref[...], preferred_element_type=jnp.float32)
        + f1b_ref[...], 0.0).astype(bf)
    o_ref[...] = (jnp.dot(h, f2_ref[...], preferred_element_type=jnp.float32)
                  + f2b_ref[...])


def kernel(x, conv1_w, conv1_b, conv2_w, conv2_b, fc1_w, fc1_b, fc2_w, fc2_b):
    N, C, S, F = x.shape
    assert C == 1 and F == _F
    n_ch = conv1_w.shape[0]
    hid = fc1_w.shape[0]
    bf = jnp.bfloat16

    npad = _round_up(max(N, 1), _M)
    nblocks = npad // _M

    xs = x.reshape(N, S * F).astype(bf)
    if npad != N:
        xs = jnp.pad(xs, ((0, npad - N), (0, 0)))

    w1_k = conv1_w.reshape(-1).astype(jnp.float32)
    b1_k = conv1_b.reshape(-1).astype(jnp.float32)
    b2_k = conv2_b.reshape(-1).astype(jnp.float32)

    bk = np.stack([np.kron(np.eye(S, k=1 - j, dtype=np.float32),
                           np.eye(F, dtype=np.float32)) for j in range(3)])
    bk = jnp.asarray(bk)
    w2m = conv2_w.reshape(n_ch, 3).astype(jnp.float32)
    t2 = jnp.einsum('cj,jpq->cpq', w2m, bk).reshape(n_ch * S * F, S * F)

    eye = jnp.eye(S, dtype=jnp.float32)
    f1 = jnp.kron(eye, fc1_w.T.astype(jnp.float32))
    f1b = jnp.tile(fc1_b.astype(jnp.float32), S).reshape(1, S * hid)

    blend = np.zeros((2 * S, S + 1), np.float32)
    blend[0, 0] = 1.0
    for s in range(1, S):
        blend[2 * s - 1, s] = 0.5
        blend[2 * s, s] = 0.5
    blend[2 * S - 1, S] = 1.0
    blend = jnp.asarray(blend)
    f2 = jnp.kron(eye, fc2_w.T.astype(jnp.float32)) @ blend
    f2b = (jnp.tile(fc2_b.astype(jnp.float32), S) @ blend).reshape(1, S + 1)

    smem = pl.BlockSpec(memory_space=pltpu.MemorySpace.SMEM)
    full = lambda r, c: pl.BlockSpec((r, c), lambda b: (0, 0))

    out = pl.pallas_call(
        functools.partial(_fused_kernel, n_ch=n_ch, grp=8),
        out_shape=jax.ShapeDtypeStruct((npad, S + 1), jnp.float32),
        grid=(nblocks,),
        in_specs=[
            pl.BlockSpec((_M, S * F), lambda b: (b, 0)),
            smem, smem,
            full(n_ch * S * F, S * F),
            smem,
            full(S * F, S * hid),
            full(1, S * hid),
            full(S * hid, S + 1),
            full(1, S + 1),
        ],
        out_specs=pl.BlockSpec((_M, S + 1), lambda b: (b, 0)),
        compiler_params=pltpu.CompilerParams(
            dimension_semantics=("parallel",),
            vmem_limit_bytes=64 * 1024 * 1024),
    )(xs, w1_k, b1_k, t2.astype(bf), b2_k, f1.astype(bf), f1b,
      f2.astype(bf), f2b)

    return out[:N]


# speedup vs baseline: 1.0836x; 1.0179x over previous
import functools

import numpy as np

import jax
import jax.numpy as jnp
from jax.experimental import pallas as pl
from jax.experimental.pallas import tpu as pltpu

_F = 12
_M = 1024


def _round_up(a, m):
    return (a + m - 1) // m * m


def _fused_kernel(x_ref, w1_ref, b1_ref, t2_ref, b2_ref,
                  f1_ref, f1b_ref, f2_ref, f2b_ref, o_ref, *, n_ch, grp):
    bf = jnp.bfloat16
    x2 = x_ref[...].astype(jnp.float32)
    m = x2.shape[0]
    sf = x2.shape[1]
    zf = jnp.zeros((m, _F), jnp.float32)

    xm = jnp.concatenate([zf, x2[:, :-_F]], axis=1)
    xp = jnp.concatenate([x2[:, _F:], zf], axis=1)

    acc = None
    for g in range(0, n_ch, grp):
        hs = []
        for c in range(g, g + grp):
            hs.append(jnp.maximum(
                w1_ref[3 * c] * xm + w1_ref[3 * c + 1] * x2
                + w1_ref[3 * c + 2] * xp + b1_ref[c], 0.0).astype(bf))
        hg = jnp.concatenate(hs, axis=1)
        p = jnp.dot(hg, t2_ref[g * sf:(g + grp) * sf, :],
                    preferred_element_type=jnp.float32)
        acc = p if g == 0 else acc + p

    y2 = jnp.maximum(acc + b2_ref[0], 0.0).astype(bf)

    h = jnp.maximum(
        jnp.dot(y2, f1_ref[...], preferred_element_type=jnp.float32)
        + f1b_ref[...], 0.0).astype(bf)
    o_ref[...] = (jnp.dot(h, f2_ref[...], preferred_element_type=jnp.float32)
                  + f2b_ref[...])


def kernel(x, conv1_w, conv1_b, conv2_w, conv2_b, fc1_w, fc1_b, fc2_w, fc2_b):
    N, C, S, F = x.shape
    assert C == 1 and F == _F
    n_ch = conv1_w.shape[0]
    hid = fc1_w.shape[0]
    bf = jnp.bfloat16

    npad = _round_up(max(N, 1), _M)
    nblocks = npad // _M

    xs = x.reshape(N, S * F).astype(bf)
    if npad != N:
        xs = jnp.pad(xs, ((0, npad - N), (0, 0)))

    w1_k = conv1_w.reshape(-1).astype(jnp.float32)
    b1_k = conv1_b.reshape(-1).astype(jnp.float32)
    b2_k = conv2_b.reshape(-1).astype(jnp.float32)

    bk = np.stack([np.kron(np.eye(S, k=1 - j, dtype=np.float32),
                           np.eye(F, dtype=np.float32)) for j in range(3)])
    bk = jnp.asarray(bk)
    w2m = conv2_w.reshape(n_ch, 3).astype(jnp.float32)
    t2 = jnp.einsum('cj,jpq->cpq', w2m, bk).reshape(n_ch * S * F, S * F)

    eye = jnp.eye(S, dtype=jnp.float32)
    f1 = jnp.kron(eye, fc1_w.T.astype(jnp.float32))
    f1b = jnp.tile(fc1_b.astype(jnp.float32), S).reshape(1, S * hid)

    blend = np.zeros((2 * S, S + 1), np.float32)
    blend[0, 0] = 1.0
    for s in range(1, S):
        blend[2 * s - 1, s] = 0.5
        blend[2 * s, s] = 0.5
    blend[2 * S - 1, S] = 1.0
    blend = jnp.asarray(blend)
    f2 = jnp.kron(eye, fc2_w.T.astype(jnp.float32)) @ blend
    f2b = (jnp.tile(fc2_b.astype(jnp.float32), S) @ blend).reshape(1, S + 1)

    smem = pl.BlockSpec(memory_space=pltpu.MemorySpace.SMEM)
    full = lambda r, c: pl.BlockSpec((r, c), lambda b: (0, 0))

    out = pl.pallas_call(
        functools.partial(_fused_kernel, n_ch=n_ch, grp=8),
        out_shape=jax.ShapeDtypeStruct((npad, S + 1), jnp.float32),
        grid=(nblocks,),
        in_specs=[
            pl.BlockSpec((_M, S * F), lambda b: (b, 0)),
            smem, smem,
            full(n_ch * S * F, S * F),
            smem,
            full(S * F, S * hid),
            full(1, S * hid),
            full(S * hid, S + 1),
            full(1, S + 1),
        ],
        out_specs=pl.BlockSpec((_M, S + 1), lambda b: (b, 0)),
        compiler_params=pltpu.CompilerParams(
            dimension_semantics=("parallel",),
            vmem_limit_bytes=64 * 1024 * 1024),
    )(xs, w1_k, b1_k, t2.astype(bf), b2_k, f1.astype(bf), f1b,
      f2.astype(bf), f2b)

    return out[:N]
